# baseline (speedup 1.0000x reference)
import functools

import jax
import jax.numpy as jnp
from jax import lax
from jax.experimental import pallas as pl
from jax.experimental.pallas import tpu as pltpu
from jax.experimental.pallas import tpu_sc as plsc

NUM_HEADS = 16
NUM_TYPES = 32
S = 2048
N = S * S

NC = 2
NS = 16
L = 16
NW = NC * NS
TROWS = S // 8
TROWS_W = TROWS // NW
CW = 1024
CHUNK = 8 * CW
N_CHUNKS = TROWS_W * (S // CW)
GROUPS = CHUNK // L
HQ = 4
NPASS = NUM_HEADS // HQ
NBUF = 2

_mesh = plsc.VectorSubcoreMesh(core_axis_name="c", subcore_axis_name="s")


@functools.partial(
    pl.kernel,
    out_type=jax.ShapeDtypeStruct((NUM_HEADS, S, S), jnp.float32),
    mesh=_mesh,
    scratch_types=[
        pltpu.VMEM((NUM_HEADS * NUM_TYPES,), jnp.float32),
        pltpu.VMEM((NBUF, 8, CW), jnp.int32),
        pltpu.VMEM((2, HQ, 8, CW), jnp.float32),
        pltpu.SemaphoreType.DMA,
        pltpu.SemaphoreType.DMA,
    ],
    compiler_params=pltpu.CompilerParams(
        needs_layout_passes=False, use_tc_tiling_on_sc=True),
)
def _edge_bias_sc(idx_hbm, tbl_hbm, out_hbm, tbl_v, idx_v, out_v, in_sem,
                  out_sem):
    wid = lax.axis_index("s") * NC + lax.axis_index("c")
    row0 = wid * TROWS_W * 8
    cpr = S // CW

    def chunk_slices(c):
        r = row0 + (c // cpr) * 8
        col = (c % cpr) * CW
        return pl.ds(r, 8), pl.ds(col, CW)

    pltpu.sync_copy(tbl_hbm, tbl_v)
    r0, c0 = chunk_slices(0)
    pltpu.async_copy(idx_hbm.at[r0, c0], idx_v.at[0], in_sem)

    def pair_body(p, carry):
        for b in range(NBUF):
            c = p * NBUF + b
            rs, cs = chunk_slices(c)
            nb = (b + 1) % NBUF

            @pl.when(c + 1 < N_CHUNKS)
            def _prefetch():
                nrs, ncs = chunk_slices(c + 1)
                pltpu.async_copy(idx_hbm.at[nrs, ncs], idx_v.at[nb], in_sem)

            pltpu.make_async_copy(idx_hbm.at[rs, cs], idx_v.at[b],
                                  in_sem).wait()

            for q in range(NPASS):
                h0 = q * HQ
                hs = pl.ds(h0, HQ)
                slot = q % 2

                if q >= 2:
                    phs = pl.ds((q - 2) * HQ, HQ)
                    pltpu.make_async_copy(out_v.at[slot],
                                          out_hbm.at[phs, rs, cs],
                                          out_sem).wait()
                else:
                    phs = pl.ds((q + 2) * HQ, HQ)

                    @pl.when(c >= 1)
                    def _drain():
                        prs, pcs = chunk_slices(c - 1)
                        pltpu.make_async_copy(out_v.at[slot],
                                              out_hbm.at[phs, prs, pcs],
                                              out_sem).wait()

                @plsc.parallel_loop(0, GROUPS, unroll=2)
                def grp_body(g):
                    row = g // (CW // L)
                    col = (g % (CW // L)) * L
                    idx = idx_v[b, row, pl.ds(col, L)]
                    for hh in range(HQ):
                        vals = plsc.load_gather(
                            tbl_v, [idx + (h0 + hh) * NUM_TYPES])
                        out_v[slot, hh, row, pl.ds(col, L)] = vals

                pltpu.async_copy(out_v.at[slot], out_hbm.at[hs, rs, cs],
                                 out_sem)
        return carry

    lax.fori_loop(0, N_CHUNKS // NBUF, pair_body, 0)
    rl, cl = chunk_slices(N_CHUNKS - 1)
    for q in range(2, NPASS):
        hs = pl.ds(q * HQ, HQ)
        pltpu.make_async_copy(out_v.at[q % 2], out_hbm.at[hs, rl, cl],
                              out_sem).wait()


def kernel(edge_type_matrix, edge_embedding_weight):
    idx = edge_type_matrix.astype(jnp.int32)
    tbl = edge_embedding_weight.T.reshape(-1)
    return _edge_bias_sc(idx, tbl)


# speedup vs baseline: 1.1392x; 1.1392x over previous
import functools

import jax
import jax.numpy as jnp
from jax import lax
from jax.experimental import pallas as pl
from jax.experimental.pallas import tpu as pltpu
from jax.experimental.pallas import tpu_sc as plsc

NUM_HEADS = 16
NUM_TYPES = 32
S = 2048
N = S * S

NC = 2
NS = 16
L = 16
NW = NC * NS
TROWS = S // 8
TROWS_W = TROWS // NW
CW = 512
CHUNK = 8 * CW
N_CHUNKS = TROWS_W * (S // CW)
GROUPS = CHUNK // L
HHALF = NUM_HEADS // 2
NBUF = 2

_mesh = plsc.VectorSubcoreMesh(core_axis_name="c", subcore_axis_name="s")


@functools.partial(
    pl.kernel,
    out_type=jax.ShapeDtypeStruct((NUM_HEADS, S, S), jnp.float32),
    mesh=_mesh,
    scratch_types=[
        pltpu.VMEM((NUM_HEADS * NUM_TYPES,), jnp.float32),
        pltpu.VMEM((NBUF, 8, CW), jnp.int32),
        pltpu.VMEM((2, HHALF, 8, CW), jnp.float32),
        pltpu.SemaphoreType.DMA,
        pltpu.SemaphoreType.DMA,
    ],
    compiler_params=pltpu.CompilerParams(
        needs_layout_passes=False, use_tc_tiling_on_sc=True),
)
def _edge_bias_sc(idx_hbm, tbl_hbm, out_hbm, tbl_v, idx_v, out_v, in_sem,
                  out_sem):
    wid = lax.axis_index("s") * NC + lax.axis_index("c")
    row0 = wid * TROWS_W * 8
    cpr = S // CW

    def chunk_slices(c):
        r = row0 + (c // cpr) * 8
        col = (c % cpr) * CW
        return pl.ds(r, 8), pl.ds(col, CW)

    pltpu.sync_copy(tbl_hbm, tbl_v)
    r0, c0 = chunk_slices(0)
    pltpu.async_copy(idx_hbm.at[r0, c0], idx_v.at[0], in_sem)

    def pair_body(p, carry):
        for b in range(NBUF):
            c = p * NBUF + b
            rs, cs = chunk_slices(c)
            nb = (b + 1) % NBUF

            @pl.when(c + 1 < N_CHUNKS)
            def _prefetch():
                nrs, ncs = chunk_slices(c + 1)
                pltpu.async_copy(idx_hbm.at[nrs, ncs], idx_v.at[nb], in_sem)

            pltpu.make_async_copy(idx_hbm.at[rs, cs], idx_v.at[b],
                                  in_sem).wait()

            for half in range(2):
                h0 = half * HHALF
                hs = pl.ds(h0, HHALF)

                @pl.when(c >= 1)
                def _drain():
                    prs, pcs = chunk_slices(c - 1)
                    pltpu.make_async_copy(out_v.at[half],
                                          out_hbm.at[hs, prs, pcs],
                                          out_sem).wait()

                @plsc.parallel_loop(0, GROUPS, unroll=4)
                def grp_body(g):
                    row = g // (CW // L)
                    col = (g % (CW // L)) * L
                    idx = idx_v[b, row, pl.ds(col, L)]
                    for hh in range(HHALF):
                        vals = plsc.load_gather(
                            tbl_v, [idx + (h0 + hh) * NUM_TYPES])
                        out_v[half, hh, row, pl.ds(col, L)] = vals

                pltpu.async_copy(out_v.at[half], out_hbm.at[hs, rs, cs],
                                 out_sem)
        return carry

    lax.fori_loop(0, N_CHUNKS // NBUF, pair_body, 0)
    rl, cl = chunk_slices(N_CHUNKS - 1)
    for half in range(2):
        hs = pl.ds(half * HHALF, HHALF)
        pltpu.make_async_copy(out_v.at[half], out_hbm.at[hs, rl, cl],
                              out_sem).wait()


def kernel(edge_type_matrix, edge_embedding_weight):
    idx = edge_type_matrix.astype(jnp.int32)
    tbl = edge_embedding_weight.T.reshape(-1)
    return _edge_bias_sc(idx, tbl)
